# baseline (speedup 1.0000x reference)
import functools

import jax
import jax.numpy as jnp
from jax import lax
from jax.experimental import pallas as pl
from jax.experimental.pallas import tpu as pltpu
from jax.experimental.pallas import tpu_sc as plsc

NUM_CORES = 2
NUM_SUBCORES = 16
NUM_TILES = NUM_CORES * NUM_SUBCORES
CB = 128


@functools.lru_cache(maxsize=None)
def _build(Bt, S, D, SP):
    bpt = Bt // NUM_TILES
    cpr = bpt // CB
    n_units = S * cpr
    assert n_units % 2 == 0 and cpr % 2 == 0
    mesh = plsc.VectorSubcoreMesh(core_axis_name="c", subcore_axis_name="s")

    def body(tok_ref, table_ref, out_ref, idx_v, rows0, rows1, g0, g1, o0, o1):
        wid = lax.axis_index("s") * NUM_CORES + lax.axis_index("c")
        b0 = wid * bpt
        pltpu.sync_copy(tok_ref.at[:, pl.ds(b0, bpt)], idx_v)
        rows = (rows0, rows1)
        gsem = (g0, g1)
        osem = (o0, o1)

        def fire_gather(u, buf):
            s = lax.div(u, cpr)
            c = lax.rem(u, cpr)
            pltpu.async_copy(
                table_ref.at[idx_v.at[s, pl.ds(c * CB, CB)]],
                rows[buf].at[:, 0], gsem[buf])

        def wait_gather(buf):
            pltpu.make_async_copy(
                table_ref.at[idx_v.at[0, pl.ds(0, CB)]],
                rows[buf].at[:, 0], gsem[buf]).wait()

        def out_slice(u):
            s = lax.div(u, cpr)
            c = lax.rem(u, cpr)
            return out_ref.at[pl.ds(b0 + c * CB, CB), pl.ds(s, 1)]

        def fire_out(u, buf):
            pltpu.async_copy(
                rows[buf].at[:, :, pl.ds(0, D)], out_slice(u), osem[buf])

        def wait_out(u, buf):
            pltpu.make_async_copy(
                rows[buf].at[:, :, pl.ds(0, D)], out_slice(u), osem[buf]).wait()

        fire_gather(0, 0)
        fire_gather(1, 1)
        wait_gather(0)
        fire_out(0, 0)

        @pl.loop(1, n_units - 1, step=2)
        def _pair(u):
            wait_out(u - 1, 0)
            fire_gather(u + 1, 0)
            wait_gather(1)
            fire_out(u, 1)

            wait_out(u, 1)
            fire_gather(u + 2, 1)
            wait_gather(0)
            fire_out(u + 1, 0)

        u_last = n_units - 1
        wait_gather(1)
        fire_out(u_last, 1)
        wait_out(u_last - 1, 0)
        wait_out(u_last, 1)

    return pl.kernel(
        body,
        out_type=jax.ShapeDtypeStruct((Bt, S, D), jnp.float32),
        mesh=mesh,
        scratch_types=[
            pltpu.VMEM((SP, Bt // NUM_TILES), jnp.int32),
            pltpu.VMEM((CB, 1, 128), jnp.float32),
            pltpu.VMEM((CB, 1, 128), jnp.float32),
            pltpu.SemaphoreType.DMA,
            pltpu.SemaphoreType.DMA,
            pltpu.SemaphoreType.DMA,
            pltpu.SemaphoreType.DMA,
        ],
        compiler_params=pltpu.CompilerParams(use_tc_tiling_on_sc=False),
    )


def kernel(token_ids, weight):
    Bt, S = token_ids.shape
    V, D = weight.shape
    SP = (S + 7) // 8 * 8
    tok = jnp.pad(token_ids.T.astype(jnp.int32), ((0, SP - S), (0, 0)))
    wpad = jnp.pad(weight, ((0, 0), (0, 128 - D)))
    return _build(Bt, S, D, SP)(tok, wpad)


# speedup vs baseline: 1.3615x; 1.3615x over previous
import functools

import jax
import jax.numpy as jnp
from jax import lax
from jax.experimental import pallas as pl
from jax.experimental.pallas import tpu as pltpu
from jax.experimental.pallas import tpu_sc as plsc

NUM_CORES = 2
NUM_SUBCORES = 16
NUM_TILES = NUM_CORES * NUM_SUBCORES
CB = 128


@functools.lru_cache(maxsize=None)
def _build(Bt, S, D, SP):
    bpt = Bt // NUM_TILES
    cpr = bpt // CB
    n_units = S * cpr
    assert n_units % 2 == 0 and cpr % 2 == 0
    mesh = plsc.VectorSubcoreMesh(core_axis_name="c", subcore_axis_name="s")

    def body(tok_ref, table_ref, out_ref, idx_v, rows0, rows1, g0, g1, o0, o1):
        wid = lax.axis_index("s") * NUM_CORES + lax.axis_index("c")
        b0 = wid * bpt
        pltpu.sync_copy(tok_ref.at[:, pl.ds(b0, bpt)], idx_v)
        rows = (rows0, rows1)
        gsem = (g0, g1)
        osem = (o0, o1)

        def fire_gather(u, buf):
            s = lax.div(u, cpr)
            c = lax.rem(u, cpr)
            pltpu.async_copy(
                table_ref.at[idx_v.at[s, pl.ds(c * CB, CB)]],
                rows[buf].at[:, 0], gsem[buf])

        def wait_gather(buf):
            pltpu.make_async_copy(
                table_ref.at[idx_v.at[0, pl.ds(0, CB)]],
                rows[buf].at[:, 0], gsem[buf]).wait()

        def out_slice(u):
            s = lax.div(u, cpr)
            c = lax.rem(u, cpr)
            return out_ref.at[pl.ds(b0 + c * CB, CB), pl.ds(s, 1),
                              pl.ds(0, D)]

        def fire_out(u, buf):
            pltpu.async_copy(
                rows[buf].at[:, :, pl.ds(0, D)], out_slice(u), osem[buf])

        def wait_out(u, buf):
            pltpu.make_async_copy(
                rows[buf].at[:, :, pl.ds(0, D)], out_slice(u), osem[buf]).wait()

        fire_gather(0, 0)
        fire_gather(1, 1)
        wait_gather(0)
        fire_out(0, 0)

        @pl.loop(1, n_units - 1, step=2)
        def _pair(u):
            wait_out(u - 1, 0)
            fire_gather(u + 1, 0)
            wait_gather(1)
            fire_out(u, 1)

            wait_out(u, 1)
            fire_gather(u + 2, 1)
            wait_gather(0)
            fire_out(u + 1, 0)

        u_last = n_units - 1
        wait_gather(1)
        fire_out(u_last, 1)
        wait_out(u_last - 1, 0)
        wait_out(u_last, 1)

    return pl.kernel(
        body,
        out_type=jax.ShapeDtypeStruct((Bt, SP, 128), jnp.float32),
        mesh=mesh,
        scratch_types=[
            pltpu.VMEM((SP, Bt // NUM_TILES), jnp.int32),
            pltpu.VMEM((CB, 1, 128), jnp.float32),
            pltpu.VMEM((CB, 1, 128), jnp.float32),
            pltpu.SemaphoreType.DMA,
            pltpu.SemaphoreType.DMA,
            pltpu.SemaphoreType.DMA,
            pltpu.SemaphoreType.DMA,
        ],
        compiler_params=pltpu.CompilerParams(use_tc_tiling_on_sc=False),
    )


def kernel(token_ids, weight):
    Bt, S = token_ids.shape
    V, D = weight.shape
    SP = (S + 7) // 8 * 8
    tok = jnp.pad(token_ids.T.astype(jnp.int32), ((0, SP - S), (0, 0)))
    wpad = jnp.pad(weight, ((0, 0), (0, 128 - D)))
    out = _build(Bt, S, D, SP)(tok, wpad)
    return out[:, :S, :D]


# speedup vs baseline: 1.3846x; 1.0170x over previous
import functools

import jax
import jax.numpy as jnp
from jax import lax
from jax.experimental import pallas as pl
from jax.experimental.pallas import tpu as pltpu
from jax.experimental.pallas import tpu_sc as plsc

NUM_CORES = 2
NUM_SUBCORES = 16
NUM_TILES = NUM_CORES * NUM_SUBCORES
CB = 128
NBUF = 4


@functools.lru_cache(maxsize=None)
def _build(Bt, S, D, SP):
    bpt = Bt // NUM_TILES
    cpr = bpt // CB
    n_units = S * cpr
    assert (n_units - 1) % NBUF == 1 % NBUF or True
    assert (n_units - NBUF) % NBUF == 0
    mesh = plsc.VectorSubcoreMesh(core_axis_name="c", subcore_axis_name="s")

    def body(tok_ref, table_ref, out_ref, idx_v, rows0, rows1, rows2, rows3,
             g0, g1, g2, g3, o0, o1, o2, o3):
        wid = lax.axis_index("s") * NUM_CORES + lax.axis_index("c")
        b0 = wid * bpt
        pltpu.sync_copy(tok_ref.at[:, pl.ds(b0, bpt)], idx_v)
        rows = (rows0, rows1, rows2, rows3)
        gsem = (g0, g1, g2, g3)
        osem = (o0, o1, o2, o3)

        def fire_gather(u, buf):
            s = lax.div(u, cpr)
            c = lax.rem(u, cpr)
            pltpu.async_copy(
                table_ref.at[idx_v.at[s, pl.ds(c * CB, CB)]],
                rows[buf].at[:, 0], gsem[buf])

        def wait_gather(buf):
            pltpu.make_async_copy(
                table_ref.at[idx_v.at[0, pl.ds(0, CB)]],
                rows[buf].at[:, 0], gsem[buf]).wait()

        def out_slice(u):
            s = lax.div(u, cpr)
            c = lax.rem(u, cpr)
            return out_ref.at[pl.ds(b0 + c * CB, CB), pl.ds(s, 1),
                              pl.ds(0, D)]

        def fire_out(u, buf):
            pltpu.async_copy(
                rows[buf].at[:, :, pl.ds(0, D)], out_slice(u), osem[buf])

        def wait_out(u, buf):
            pltpu.make_async_copy(
                rows[buf].at[:, :, pl.ds(0, D)], out_slice(u), osem[buf]).wait()

        def step(u, k, first=False, fire=True):
            if fire:
                if not first:
                    wait_out(u - 1, (k - 1) % NBUF)
                fire_gather(u + NBUF - 1, (k - 1) % NBUF)
            wait_gather(k)
            fire_out(u, k)

        for k in range(NBUF - 1):
            fire_gather(k, k)
        step(0, 0, first=True)

        @pl.loop(1, n_units - NBUF + 1, step=NBUF)
        def _round(u):
            for k in range(NBUF):
                step(u + k, (1 + k) % NBUF)

        for k in range(NBUF - 1):
            u = n_units - NBUF + 1 + k
            step(u, u % NBUF, fire=False)
        for k in range(NBUF):
            u = n_units - NBUF + k
            wait_out(u, u % NBUF)

    return pl.kernel(
        body,
        out_type=jax.ShapeDtypeStruct((Bt, SP, 128), jnp.float32),
        mesh=mesh,
        scratch_types=[
            pltpu.VMEM((SP, Bt // NUM_TILES), jnp.int32),
            pltpu.VMEM((CB, 1, 128), jnp.float32),
            pltpu.VMEM((CB, 1, 128), jnp.float32),
            pltpu.VMEM((CB, 1, 128), jnp.float32),
            pltpu.VMEM((CB, 1, 128), jnp.float32),
            pltpu.SemaphoreType.DMA,
            pltpu.SemaphoreType.DMA,
            pltpu.SemaphoreType.DMA,
            pltpu.SemaphoreType.DMA,
            pltpu.SemaphoreType.DMA,
            pltpu.SemaphoreType.DMA,
            pltpu.SemaphoreType.DMA,
            pltpu.SemaphoreType.DMA,
        ],
        compiler_params=pltpu.CompilerParams(use_tc_tiling_on_sc=False),
    )


def kernel(token_ids, weight):
    Bt, S = token_ids.shape
    V, D = weight.shape
    SP = (S + 7) // 8 * 8
    tok = jnp.pad(token_ids.T.astype(jnp.int32), ((0, SP - S), (0, 0)))
    wpad = jnp.pad(weight, ((0, 0), (0, 128 - D)))
    out = _build(Bt, S, D, SP)(tok, wpad)
    return out[:, :S, :D]
